# baseline (speedup 1.0000x reference)
import functools

import jax
import jax.numpy as jnp
from jax import lax
from jax.experimental import pallas as pl
from jax.experimental.pallas import tpu as pltpu
from jax.experimental.pallas import tpu_sc as plsc

N = 50000
E = 800000
HEADS = 4
HID = 16
OUT_DIM = 6

CHUNK = 12500
NRANGE = 2
ACC_ROWS = 12544
EB = 128
EP = 851968
NTILES = 16
EDGES_PER_TILE = EP // NTILES
NBLK = EDGES_PER_TILE // EB
ROWS_PER_TILE = ACC_ROWS // NTILES
ZROWS = 16
BT = 2000
GRID = N // BT


def _leaky_exp(a, b):
    s = a + b
    return jnp.exp(jnp.where(s > 0, s, 0.2 * s))


def _make_edge_pass(width, nheads, nbuf):
    mesh = plsc.VectorSubcoreMesh(core_axis_name="c", subcore_axis_name="s")
    nsteps = NBLK // nbuf

    slot_scratch = [
        pltpu.VMEM((EB,), jnp.int32),
        pltpu.VMEM((EB,), jnp.int32),
        pltpu.VMEM((EB,), jnp.int32),
        pltpu.VMEM((EB,), jnp.int32),
        pltpu.VMEM((EB,), jnp.int32),
        pltpu.VMEM((EB, width), jnp.float32),
        pltpu.VMEM((EB, 16), jnp.float32),
        pltpu.VMEM((EB, width), jnp.float32),
        pltpu.SemaphoreType.DMA,
        pltpu.SemaphoreType.DMA,
    ]
    nslot = len(slot_scratch)

    @functools.partial(
        pl.kernel,
        mesh=mesh,
        out_type=jax.ShapeDtypeStruct((2 * NRANGE, ACC_ROWS, width),
                                      jnp.float32),
        compiler_params=pltpu.CompilerParams(use_tc_tiling_on_sc=False),
        scratch_types=(slot_scratch * nbuf) + [
            pltpu.VMEM((ZROWS, width), jnp.float32),
            pltpu.VMEM_SHARED((ACC_ROWS, width), jnp.float32),
        ],
    )
    def edge_pass(tabs, tabd, srcs, dsts, out_hbm, *refs):
        slots = [refs[i * nslot:(i + 1) * nslot] for i in range(nbuf)]
        zbuf, acc = refs[nbuf * nslot], refs[nbuf * nslot + 1]
        c = lax.axis_index("c")
        s = lax.axis_index("s")
        z16 = jnp.zeros((16,), jnp.float32)

        def zrow(r, _):
            for k in range(width // 16):
                zbuf[r, pl.ds(16 * k, 16)] = z16
            return _
        lax.fori_loop(0, ZROWS, zrow, None)

        rowbase = s * ROWS_PER_TILE
        ebase = s * EDGES_PER_TILE
        lane = lax.iota(jnp.int32, 16)

        def issue_ids(b, blk):
            ids_s, ids_d, _, _, _, _, _, _, _, isem = slots[b]
            estart = ebase + blk * EB
            pltpu.async_copy(srcs.at[pl.ds(estart, EB)], ids_s, isem)
            pltpu.async_copy(dsts.at[pl.ds(estart, EB)], ids_d, isem)

        def wait_ids(b):
            ids_s, ids_d, _, _, _, _, _, _, _, isem = slots[b]
            pltpu.make_async_copy(srcs.at[pl.ds(0, EB)], ids_s, isem).wait()
            pltpu.make_async_copy(dsts.at[pl.ds(0, EB)], ids_d, isem).wait()

        def issue_gathers(b):
            _, _, gsx, gdx, _, srows, drows, _, gsem, _ = slots[b]
            pltpu.async_copy(tabs.at[gsx], srows, gsem)
            pltpu.async_copy(tabd.at[gdx], drows, gsem)

        def wait_gathers(b):
            _, _, gsx, gdx, _, srows, drows, _, gsem, _ = slots[b]
            pltpu.make_async_copy(tabs.at[gsx], srows, gsem).wait()
            pltpu.make_async_copy(tabd.at[gdx], drows, gsem).wait()

        def unpack(b, base):
            ids_s, ids_d, gsx, gdx, lidx, _, _, _, _, _ = slots[b]
            for g in range(EB // 16):
                s16 = ids_s[pl.ds(16 * g, 16)]
                d16 = ids_d[pl.ds(16 * g, 16)]
                dl = d16 - base
                own = (dl >= 0) & (dl < CHUNK)
                gsx[pl.ds(16 * g, 16)] = s16
                gdx[pl.ds(16 * g, 16)] = jnp.where(own, d16, base + CHUNK)
                lidx[pl.ds(16 * g, 16)] = jnp.where(own, dl, CHUNK)

        def compute_and_scatter(b):
            _, _, _, _, lidx, srows, drows, msg, _, _ = slots[b]

            def edge_body(e, _):
                if nheads == 4:
                    ea = srows[e, pl.ds(64, 16)]
                    eb = drows[e, pl.ds(0, 16)]
                    exv = _leaky_exp(ea, eb)
                    msg[e, pl.ds(64, 16)] = jnp.where(lane < 4, exv, 0.0)
                    for h in range(4):
                        msg[e, pl.ds(16 * h, 16)] = (
                            srows[e, pl.ds(16 * h, 16)] * exv[h])
                else:
                    av = srows[e, pl.ds(0, 16)]
                    bv = drows[e, pl.ds(0, 16)]
                    sc = av[7] + bv[0]
                    sc = jnp.where(sc > 0, sc, 0.2 * sc)
                    exv = jnp.exp(jnp.broadcast_to(sc, (16,)))
                    msg[e, pl.ds(0, 16)] = av * exv
                return _
            lax.fori_loop(0, EB, edge_body, None)
            pltpu.sync_copy(msg, acc.at[lidx], add=True)

        for rng in range(NRANGE):
            slab = c * NRANGE + rng
            base = slab * CHUNK

            def zacc(i, _):
                pltpu.sync_copy(zbuf,
                                acc.at[pl.ds(rowbase + i * ZROWS, ZROWS)])
                return _
            lax.fori_loop(0, ROWS_PER_TILE // ZROWS, zacc, None)

            plsc.subcore_barrier()

            def step(blk, _):
                b = 0
                ids_s, ids_d = slots[b][0], slots[b][1]
                estart = ebase + blk * EB
                pltpu.sync_copy(srcs.at[pl.ds(estart, EB)], ids_s)
                pltpu.sync_copy(dsts.at[pl.ds(estart, EB)], ids_d)
                unpack(b, base)
                issue_gathers(b)
                wait_gathers(b)
                compute_and_scatter(b)
                return _
            lax.fori_loop(0, NBLK, step, None)

            plsc.subcore_barrier()
            pltpu.sync_copy(acc.at[pl.ds(rowbase, ROWS_PER_TILE)],
                            out_hbm.at[slab, pl.ds(rowbase, ROWS_PER_TILE)])

    return edge_pass


_edge_pass_cache = {}


def _edge_pass(width, nheads, nbuf):
    key = (width, nheads)
    if key not in _edge_pass_cache:
        _edge_pass_cache[key] = _make_edge_pass(width, nheads, nbuf)
    return _edge_pass_cache[key]


def _dense1_body(x_ref, w_ref, as_ref, ad_ref, ts_ref, td_ref):
    h = jnp.dot(x_ref[...], w_ref[...], preferred_element_type=jnp.float32)
    asv = as_ref[0:1, :]
    adv = ad_ref[0:1, :]
    acols = []
    dcols = []
    for hh in range(HEADS):
        sl = slice(16 * hh, 16 * (hh + 1))
        acols.append(jnp.sum(h[:, sl] * asv[:, sl], axis=1, keepdims=True))
        dcols.append(jnp.sum(h[:, sl] * adv[:, sl], axis=1, keepdims=True))
    z12 = jnp.zeros((BT, 12), jnp.float32)
    ts_ref[...] = jnp.concatenate([h] + acols + [z12], axis=1)
    td_ref[...] = jnp.concatenate(dcols + [z12], axis=1)


def _dense2_body(un_ref, den_ref, b1_ref, w2_ref, as_ref, ad_ref,
                 ts_ref, td_ref):
    den = den_ref[...]
    rep = jnp.concatenate(
        [jnp.broadcast_to(den[:, h:h + 1], (BT, 16)) for h in range(HEADS)],
        axis=1)
    hin = un_ref[...] / rep + b1_ref[0:1, :]
    hin = jnp.where(hin > 0, hin, jnp.exp(jnp.minimum(hin, 0.0)) - 1.0)
    h2 = jnp.dot(hin, w2_ref[...], preferred_element_type=jnp.float32)
    a2s = jnp.sum(h2 * as_ref[0:1, :], axis=1, keepdims=True)
    a2d = jnp.sum(h2 * ad_ref[0:1, :], axis=1, keepdims=True)
    ones = jnp.ones((BT, 1), jnp.float32)
    ts_ref[...] = jnp.concatenate(
        [h2[:, 0:6], ones, a2s, jnp.zeros((BT, 8), jnp.float32)], axis=1)
    td_ref[...] = jnp.concatenate(
        [a2d, jnp.zeros((BT, 15), jnp.float32)], axis=1)


def _final_body(y_ref, b2_ref, o_ref):
    y = y_ref[...]
    den = jnp.broadcast_to(y[:, 6:7], (BT, 6))
    r = y[:, 0:6] / den + b2_ref[0:1, 0:6]
    o_ref[...] = 1.0 / (1.0 + jnp.exp(-r))


def kernel(x, edge_index, W1, att_src1, att_dst1, b1, W2, att_src2,
           att_dst2, b2):
    f32 = jnp.float32
    ei = edge_index.astype(jnp.int32)
    loop = jnp.arange(N, dtype=jnp.int32)
    pad = EP - (E + N)
    srcs = jnp.concatenate([ei[0], loop, jnp.zeros((pad,), jnp.int32)])
    dsts = jnp.concatenate([ei[1], loop, jnp.full((pad,), N, jnp.int32)])

    xp = jnp.pad(x.astype(f32), ((0, 0), (0, 2)))
    w1p = jnp.pad(W1.astype(f32), ((0, 2), (0, 0)))
    as1 = jnp.pad(att_src1.astype(f32).reshape(1, 64), ((0, 7), (0, 0)))
    ad1 = jnp.pad(att_dst1.astype(f32).reshape(1, 64), ((0, 7), (0, 0)))

    tabs1, tabd1 = pl.pallas_call(
        _dense1_body,
        grid=(GRID,),
        in_specs=[
            pl.BlockSpec((BT, 8), lambda i: (i, 0)),
            pl.BlockSpec((8, 64), lambda i: (0, 0)),
            pl.BlockSpec((8, 64), lambda i: (0, 0)),
            pl.BlockSpec((8, 64), lambda i: (0, 0)),
        ],
        out_specs=[
            pl.BlockSpec((BT, 80), lambda i: (i, 0)),
            pl.BlockSpec((BT, 16), lambda i: (i, 0)),
        ],
        out_shape=[
            jax.ShapeDtypeStruct((N, 80), f32),
            jax.ShapeDtypeStruct((N, 16), f32),
        ],
    )(xp, w1p, as1, ad1)

    acc1 = _edge_pass(80, 4, 1)(
        tabs1, jnp.pad(tabd1, ((0, 16), (0, 0))), srcs, dsts)
    part1 = jnp.concatenate(
        [acc1[k, :CHUNK] for k in range(2 * NRANGE)], axis=0)
    un1 = part1[:, 0:64]
    den1 = part1[:, 64:68]

    b1p = jnp.pad(b1.astype(f32).reshape(1, 64), ((0, 7), (0, 0)))
    w2p = jnp.pad(W2.astype(f32), ((0, 0), (0, 16 - OUT_DIM)))
    as2 = jnp.pad(att_src2.astype(f32).reshape(1, OUT_DIM),
                  ((0, 7), (0, 16 - OUT_DIM)))
    ad2 = jnp.pad(att_dst2.astype(f32).reshape(1, OUT_DIM),
                  ((0, 7), (0, 16 - OUT_DIM)))

    tabs2, tabd2 = pl.pallas_call(
        _dense2_body,
        grid=(GRID,),
        in_specs=[
            pl.BlockSpec((BT, 64), lambda i: (i, 0)),
            pl.BlockSpec((BT, 4), lambda i: (i, 0)),
            pl.BlockSpec((8, 64), lambda i: (0, 0)),
            pl.BlockSpec((64, 16), lambda i: (0, 0)),
            pl.BlockSpec((8, 16), lambda i: (0, 0)),
            pl.BlockSpec((8, 16), lambda i: (0, 0)),
        ],
        out_specs=[
            pl.BlockSpec((BT, 16), lambda i: (i, 0)),
            pl.BlockSpec((BT, 16), lambda i: (i, 0)),
        ],
        out_shape=[
            jax.ShapeDtypeStruct((N, 16), f32),
            jax.ShapeDtypeStruct((N, 16), f32),
        ],
    )(un1, den1, b1p, w2p, as2, ad2)

    acc2 = _edge_pass(16, 1, 1)(
        tabs2, jnp.pad(tabd2, ((0, 16), (0, 0))), srcs, dsts)
    y = jnp.concatenate(
        [acc2[k, :CHUNK] for k in range(2 * NRANGE)], axis=0)

    b2p = jnp.pad(b2.astype(f32).reshape(1, OUT_DIM),
                  ((0, 7), (0, 16 - OUT_DIM)))
    out = pl.pallas_call(
        _final_body,
        grid=(GRID,),
        in_specs=[
            pl.BlockSpec((BT, 16), lambda i: (i, 0)),
            pl.BlockSpec((8, 16), lambda i: (0, 0)),
        ],
        out_specs=pl.BlockSpec((BT, OUT_DIM), lambda i: (i, 0)),
        out_shape=jax.ShapeDtypeStruct((N, OUT_DIM), f32),
    )(y, b2p)
    return out


# speedup vs baseline: 2.5395x; 2.5395x over previous
import functools

import jax
import jax.numpy as jnp
from jax import lax
from jax.experimental import pallas as pl
from jax.experimental.pallas import tpu as pltpu
from jax.experimental.pallas import tpu_sc as plsc

N = 50000
E = 800000
HEADS = 4
HID = 16
OUT_DIM = 6

CHUNK = 12500
NRANGE = 2
ACC_ROWS = 12544
EB = 128
EP = 851968
NTILES = 16
EDGES_PER_TILE = EP // NTILES
NBLK = EDGES_PER_TILE // EB
ROWS_PER_TILE = ACC_ROWS // NTILES
ZROWS = 16
BT = 2000
GRID = N // BT


def _leaky_exp(a, b):
    s = a + b
    return jnp.exp(jnp.where(s > 0, s, 0.2 * s))


def _make_edge_pass(width, nheads, nbuf):
    mesh = plsc.VectorSubcoreMesh(core_axis_name="c", subcore_axis_name="s")
    nsteps = NBLK // nbuf

    slot_scratch = [
        pltpu.VMEM((EB,), jnp.int32),
        pltpu.VMEM((EB,), jnp.int32),
        pltpu.VMEM((EB,), jnp.int32),
        pltpu.VMEM((EB,), jnp.int32),
        pltpu.VMEM((EB,), jnp.int32),
        pltpu.VMEM((EB, width), jnp.float32),
        pltpu.VMEM((EB, 16), jnp.float32),
        pltpu.VMEM((EB, width), jnp.float32),
        pltpu.SemaphoreType.DMA,
        pltpu.SemaphoreType.DMA,
    ]
    nslot = len(slot_scratch)

    @functools.partial(
        pl.kernel,
        mesh=mesh,
        out_type=jax.ShapeDtypeStruct((2 * NRANGE, ACC_ROWS, width),
                                      jnp.float32),
        compiler_params=pltpu.CompilerParams(use_tc_tiling_on_sc=False),
        scratch_types=(slot_scratch * nbuf) + [
            pltpu.VMEM((ZROWS, width), jnp.float32),
            pltpu.VMEM_SHARED((ACC_ROWS, width), jnp.float32),
        ],
    )
    def edge_pass(tabs, tabd, srcs, dsts, out_hbm, *refs):
        slots = [refs[i * nslot:(i + 1) * nslot] for i in range(nbuf)]
        zbuf, acc = refs[nbuf * nslot], refs[nbuf * nslot + 1]
        c = lax.axis_index("c")
        s = lax.axis_index("s")
        z16 = jnp.zeros((16,), jnp.float32)

        def zrow(r, _):
            for k in range(width // 16):
                zbuf[r, pl.ds(16 * k, 16)] = z16
            return _
        lax.fori_loop(0, ZROWS, zrow, None)

        rowbase = s * ROWS_PER_TILE
        ebase = s * EDGES_PER_TILE
        lane = lax.iota(jnp.int32, 16)

        def issue_ids(b, blk):
            ids_s, ids_d, _, _, _, _, _, _, _, isem = slots[b]
            estart = ebase + blk * EB
            pltpu.async_copy(srcs.at[pl.ds(estart, EB)], ids_s, isem)
            pltpu.async_copy(dsts.at[pl.ds(estart, EB)], ids_d, isem)

        def wait_ids(b):
            ids_s, ids_d, _, _, _, _, _, _, _, isem = slots[b]
            pltpu.make_async_copy(srcs.at[pl.ds(0, EB)], ids_s, isem).wait()
            pltpu.make_async_copy(dsts.at[pl.ds(0, EB)], ids_d, isem).wait()

        def issue_gathers(b):
            _, _, gsx, gdx, _, srows, drows, _, gsem, _ = slots[b]
            pltpu.async_copy(tabs.at[gsx], srows, gsem)
            pltpu.async_copy(tabd.at[gdx], drows, gsem)

        def wait_gathers(b):
            _, _, gsx, gdx, _, srows, drows, _, gsem, _ = slots[b]
            pltpu.make_async_copy(tabs.at[gsx], srows, gsem).wait()
            pltpu.make_async_copy(tabd.at[gdx], drows, gsem).wait()

        def unpack(b, base):
            ids_s, ids_d, gsx, gdx, lidx, _, _, _, _, _ = slots[b]
            for g in range(EB // 16):
                s16 = ids_s[pl.ds(16 * g, 16)]
                d16 = ids_d[pl.ds(16 * g, 16)]
                dl = d16 - base
                own = (dl >= 0) & (dl < CHUNK)
                gsx[pl.ds(16 * g, 16)] = s16
                gdx[pl.ds(16 * g, 16)] = jnp.where(own, d16, base + CHUNK)
                lidx[pl.ds(16 * g, 16)] = jnp.where(own, dl, CHUNK)

        def compute_and_scatter(b):
            _, _, _, _, lidx, srows, drows, msg, _, _ = slots[b]

            def edge_body(e, _):
                if nheads == 4:
                    ea = srows[e, pl.ds(64, 16)]
                    eb = drows[e, pl.ds(0, 16)]
                    exv = _leaky_exp(ea, eb)
                    msg[e, pl.ds(64, 16)] = jnp.where(lane < 4, exv, 0.0)
                    for h in range(4):
                        msg[e, pl.ds(16 * h, 16)] = (
                            srows[e, pl.ds(16 * h, 16)] * exv[h])
                else:
                    av = srows[e, pl.ds(0, 16)]
                    bv = drows[e, pl.ds(0, 16)]
                    sc = av[7] + bv[0]
                    sc = jnp.where(sc > 0, sc, 0.2 * sc)
                    exv = jnp.exp(jnp.broadcast_to(sc, (16,)))
                    msg[e, pl.ds(0, 16)] = av * exv
                return _
            lax.fori_loop(0, EB, edge_body, None)
            pltpu.sync_copy(msg, acc.at[lidx], add=True)

        for rng in range(NRANGE):
            slab = c * NRANGE + rng
            base = slab * CHUNK

            def zacc(i, _):
                pltpu.sync_copy(zbuf,
                                acc.at[pl.ds(rowbase + i * ZROWS, ZROWS)])
                return _
            lax.fori_loop(0, ROWS_PER_TILE // ZROWS, zacc, None)

            plsc.subcore_barrier()

            def step(blk, _):
                (ids_s, ids_d, gsx, gdx, lidx, srows, drows, msg,
                 gsem, isem) = slots[0]
                estart = ebase + blk * EB
                pltpu.sync_copy(srcs.at[pl.ds(estart, EB)], ids_s)
                pltpu.sync_copy(dsts.at[pl.ds(estart, EB)], ids_d)
                cp1 = pltpu.async_copy(tabs.at[ids_s], srows, gsem)
                cp2 = pltpu.async_copy(tabd.at[ids_d], drows, isem)
                for g in range(EB // 16):
                    d16 = ids_d[pl.ds(16 * g, 16)]
                    dl = d16 - base
                    own = (dl >= 0) & (dl < CHUNK)
                    lidx[pl.ds(16 * g, 16)] = jnp.where(own, dl, CHUNK)
                cp1.wait()
                cp2.wait()
                compute_and_scatter(0)
                return _
            lax.fori_loop(0, NBLK, step, None)

            plsc.subcore_barrier()
            pltpu.sync_copy(acc.at[pl.ds(rowbase, ROWS_PER_TILE)],
                            out_hbm.at[slab, pl.ds(rowbase, ROWS_PER_TILE)])

    return edge_pass


_edge_pass_cache = {}


def _edge_pass(width, nheads, nbuf):
    key = (width, nheads)
    if key not in _edge_pass_cache:
        _edge_pass_cache[key] = _make_edge_pass(width, nheads, nbuf)
    return _edge_pass_cache[key]


def _dense1_body(x_ref, w_ref, as_ref, ad_ref, ts_ref, td_ref):
    h = jnp.dot(x_ref[...], w_ref[...], preferred_element_type=jnp.float32)
    asv = as_ref[0:1, :]
    adv = ad_ref[0:1, :]
    acols = []
    dcols = []
    for hh in range(HEADS):
        sl = slice(16 * hh, 16 * (hh + 1))
        acols.append(jnp.sum(h[:, sl] * asv[:, sl], axis=1, keepdims=True))
        dcols.append(jnp.sum(h[:, sl] * adv[:, sl], axis=1, keepdims=True))
    z12 = jnp.zeros((BT, 12), jnp.float32)
    ts_ref[...] = jnp.concatenate([h] + acols + [z12], axis=1)
    td_ref[...] = jnp.concatenate(dcols + [z12], axis=1)


def _dense2_body(un_ref, den_ref, b1_ref, w2_ref, as_ref, ad_ref,
                 ts_ref, td_ref):
    den = den_ref[...]
    rep = jnp.concatenate(
        [jnp.broadcast_to(den[:, h:h + 1], (BT, 16)) for h in range(HEADS)],
        axis=1)
    hin = un_ref[...] / rep + b1_ref[0:1, :]
    hin = jnp.where(hin > 0, hin, jnp.exp(jnp.minimum(hin, 0.0)) - 1.0)
    h2 = jnp.dot(hin, w2_ref[...], preferred_element_type=jnp.float32)
    a2s = jnp.sum(h2 * as_ref[0:1, :], axis=1, keepdims=True)
    a2d = jnp.sum(h2 * ad_ref[0:1, :], axis=1, keepdims=True)
    ones = jnp.ones((BT, 1), jnp.float32)
    ts_ref[...] = jnp.concatenate(
        [h2[:, 0:6], ones, a2s, jnp.zeros((BT, 8), jnp.float32)], axis=1)
    td_ref[...] = jnp.concatenate(
        [a2d, jnp.zeros((BT, 15), jnp.float32)], axis=1)


def _final_body(y_ref, b2_ref, o_ref):
    y = y_ref[...]
    den = jnp.broadcast_to(y[:, 6:7], (BT, 6))
    r = y[:, 0:6] / den + b2_ref[0:1, 0:6]
    o_ref[...] = 1.0 / (1.0 + jnp.exp(-r))


def kernel(x, edge_index, W1, att_src1, att_dst1, b1, W2, att_src2,
           att_dst2, b2):
    f32 = jnp.float32
    ei = edge_index.astype(jnp.int32)
    loop = jnp.arange(N, dtype=jnp.int32)
    pad = EP - (E + N)
    srcs = jnp.concatenate([ei[0], loop, jnp.zeros((pad,), jnp.int32)])
    dsts = jnp.concatenate([ei[1], loop, jnp.full((pad,), N, jnp.int32)])

    xp = jnp.pad(x.astype(f32), ((0, 0), (0, 2)))
    w1p = jnp.pad(W1.astype(f32), ((0, 2), (0, 0)))
    as1 = jnp.pad(att_src1.astype(f32).reshape(1, 64), ((0, 7), (0, 0)))
    ad1 = jnp.pad(att_dst1.astype(f32).reshape(1, 64), ((0, 7), (0, 0)))

    tabs1, tabd1 = pl.pallas_call(
        _dense1_body,
        grid=(GRID,),
        in_specs=[
            pl.BlockSpec((BT, 8), lambda i: (i, 0)),
            pl.BlockSpec((8, 64), lambda i: (0, 0)),
            pl.BlockSpec((8, 64), lambda i: (0, 0)),
            pl.BlockSpec((8, 64), lambda i: (0, 0)),
        ],
        out_specs=[
            pl.BlockSpec((BT, 80), lambda i: (i, 0)),
            pl.BlockSpec((BT, 16), lambda i: (i, 0)),
        ],
        out_shape=[
            jax.ShapeDtypeStruct((N, 80), f32),
            jax.ShapeDtypeStruct((N, 16), f32),
        ],
    )(xp, w1p, as1, ad1)

    acc1 = _edge_pass(80, 4, 1)(
        tabs1, jnp.pad(tabd1, ((0, 16), (0, 0))), srcs, dsts)
    part1 = jnp.concatenate(
        [acc1[k, :CHUNK] for k in range(2 * NRANGE)], axis=0)
    un1 = part1[:, 0:64]
    den1 = part1[:, 64:68]

    b1p = jnp.pad(b1.astype(f32).reshape(1, 64), ((0, 7), (0, 0)))
    w2p = jnp.pad(W2.astype(f32), ((0, 0), (0, 16 - OUT_DIM)))
    as2 = jnp.pad(att_src2.astype(f32).reshape(1, OUT_DIM),
                  ((0, 7), (0, 16 - OUT_DIM)))
    ad2 = jnp.pad(att_dst2.astype(f32).reshape(1, OUT_DIM),
                  ((0, 7), (0, 16 - OUT_DIM)))

    tabs2, tabd2 = pl.pallas_call(
        _dense2_body,
        grid=(GRID,),
        in_specs=[
            pl.BlockSpec((BT, 64), lambda i: (i, 0)),
            pl.BlockSpec((BT, 4), lambda i: (i, 0)),
            pl.BlockSpec((8, 64), lambda i: (0, 0)),
            pl.BlockSpec((64, 16), lambda i: (0, 0)),
            pl.BlockSpec((8, 16), lambda i: (0, 0)),
            pl.BlockSpec((8, 16), lambda i: (0, 0)),
        ],
        out_specs=[
            pl.BlockSpec((BT, 16), lambda i: (i, 0)),
            pl.BlockSpec((BT, 16), lambda i: (i, 0)),
        ],
        out_shape=[
            jax.ShapeDtypeStruct((N, 16), f32),
            jax.ShapeDtypeStruct((N, 16), f32),
        ],
    )(un1, den1, b1p, w2p, as2, ad2)

    acc2 = _edge_pass(16, 1, 1)(
        tabs2, jnp.pad(tabd2, ((0, 16), (0, 0))), srcs, dsts)
    y = jnp.concatenate(
        [acc2[k, :CHUNK] for k in range(2 * NRANGE)], axis=0)

    b2p = jnp.pad(b2.astype(f32).reshape(1, OUT_DIM),
                  ((0, 7), (0, 16 - OUT_DIM)))
    out = pl.pallas_call(
        _final_body,
        grid=(GRID,),
        in_specs=[
            pl.BlockSpec((BT, 16), lambda i: (i, 0)),
            pl.BlockSpec((8, 16), lambda i: (0, 0)),
        ],
        out_specs=pl.BlockSpec((BT, OUT_DIM), lambda i: (i, 0)),
        out_shape=jax.ShapeDtypeStruct((N, OUT_DIM), f32),
    )(y, b2p)
    return out


# speedup vs baseline: 3.1606x; 1.2446x over previous
import functools

import jax
import jax.numpy as jnp
from jax import lax
from jax.experimental import pallas as pl
from jax.experimental.pallas import tpu as pltpu
from jax.experimental.pallas import tpu_sc as plsc

N = 50000
E = 800000
HEADS = 4
HID = 16
OUT_DIM = 6

CHUNK = 12500
NRANGE = 2
ACC_ROWS = 12544
EB = 128
EP = 851968
NTILES = 16
EDGES_PER_TILE = EP // NTILES
NBLK = EDGES_PER_TILE // EB
ROWS_PER_TILE = ACC_ROWS // NTILES
ZROWS = 16
BT = 2000
GRID = N // BT


def _leaky_exp(a, b):
    s = a + b
    return jnp.exp(jnp.where(s > 0, s, 0.2 * s))


def _make_edge_pass(width, nheads, nbuf):
    mesh = plsc.VectorSubcoreMesh(core_axis_name="c", subcore_axis_name="s")
    nsteps = NBLK // nbuf

    slot_scratch = [
        pltpu.VMEM((EB,), jnp.int32),
        pltpu.VMEM((EB,), jnp.int32),
        pltpu.VMEM((EB,), jnp.int32),
        pltpu.VMEM((EB,), jnp.int32),
        pltpu.VMEM((EB,), jnp.int32),
        pltpu.VMEM((EB, width), jnp.float32),
        pltpu.VMEM((EB, 16), jnp.float32),
        pltpu.VMEM((EB, width), jnp.float32),
        pltpu.SemaphoreType.DMA,
        pltpu.SemaphoreType.DMA,
    ]
    nslot = len(slot_scratch)

    @functools.partial(
        pl.kernel,
        mesh=mesh,
        out_type=jax.ShapeDtypeStruct((2 * NRANGE, ACC_ROWS, width),
                                      jnp.float32),
        compiler_params=pltpu.CompilerParams(use_tc_tiling_on_sc=False),
        scratch_types=(slot_scratch * nbuf) + [
            pltpu.VMEM((ZROWS, width), jnp.float32),
            pltpu.VMEM_SHARED((ACC_ROWS, width), jnp.float32),
        ],
    )
    def edge_pass(tabs, tabd, srcs, dsts, out_hbm, *refs):
        slots = [refs[i * nslot:(i + 1) * nslot] for i in range(nbuf)]
        zbuf, acc = refs[nbuf * nslot], refs[nbuf * nslot + 1]
        c = lax.axis_index("c")
        s = lax.axis_index("s")
        z16 = jnp.zeros((16,), jnp.float32)

        def zrow(r, _):
            for k in range(width // 16):
                zbuf[r, pl.ds(16 * k, 16)] = z16
            return _
        lax.fori_loop(0, ZROWS, zrow, None)

        rowbase = s * ROWS_PER_TILE
        ebase = s * EDGES_PER_TILE
        lane = lax.iota(jnp.int32, 16)

        def issue_ids(b, blk):
            ids_s, ids_d, _, _, _, _, _, _, _, isem = slots[b]
            estart = ebase + blk * EB
            pltpu.async_copy(srcs.at[pl.ds(estart, EB)], ids_s, isem)
            pltpu.async_copy(dsts.at[pl.ds(estart, EB)], ids_d, isem)

        def wait_ids(b):
            ids_s, ids_d, _, _, _, _, _, _, _, isem = slots[b]
            pltpu.make_async_copy(srcs.at[pl.ds(0, EB)], ids_s, isem).wait()
            pltpu.make_async_copy(dsts.at[pl.ds(0, EB)], ids_d, isem).wait()

        def issue_gathers(b):
            _, _, gsx, gdx, _, srows, drows, _, gsem, _ = slots[b]
            pltpu.async_copy(tabs.at[gsx], srows, gsem)
            pltpu.async_copy(tabd.at[gdx], drows, gsem)

        def wait_gathers(b):
            _, _, gsx, gdx, _, srows, drows, _, gsem, _ = slots[b]
            pltpu.make_async_copy(tabs.at[gsx], srows, gsem).wait()
            pltpu.make_async_copy(tabd.at[gdx], drows, gsem).wait()

        def unpack(b, base):
            ids_s, ids_d, gsx, gdx, lidx, _, _, _, _, _ = slots[b]
            for g in range(EB // 16):
                s16 = ids_s[pl.ds(16 * g, 16)]
                d16 = ids_d[pl.ds(16 * g, 16)]
                dl = d16 - base
                own = (dl >= 0) & (dl < CHUNK)
                gsx[pl.ds(16 * g, 16)] = s16
                gdx[pl.ds(16 * g, 16)] = jnp.where(own, d16, base + CHUNK)
                lidx[pl.ds(16 * g, 16)] = jnp.where(own, dl, CHUNK)

        def compute_and_scatter(b):
            _, _, _, _, lidx, srows, drows, msg, _, _ = slots[b]

            def edge_body(e, _):
                if nheads == 4:
                    ea = srows[e, pl.ds(64, 16)]
                    eb = drows[e, pl.ds(0, 16)]
                    exv = _leaky_exp(ea, eb)
                    msg[e, pl.ds(64, 16)] = jnp.where(lane < 4, exv, 0.0)
                    for h in range(4):
                        msg[e, pl.ds(16 * h, 16)] = (
                            srows[e, pl.ds(16 * h, 16)] * exv[h])
                else:
                    av = srows[e, pl.ds(0, 16)]
                    bv = drows[e, pl.ds(0, 16)]
                    sc = av[7] + bv[0]
                    sc = jnp.where(sc > 0, sc, 0.2 * sc)
                    exv = jnp.exp(jnp.broadcast_to(sc, (16,)))
                    msg[e, pl.ds(0, 16)] = av * exv
                return _
            lax.fori_loop(0, EB, edge_body, None)
            pltpu.sync_copy(msg, acc.at[lidx], add=True)

        for rng in range(NRANGE):
            slab = c * NRANGE + rng
            base = slab * CHUNK

            def zacc(i, _):
                pltpu.sync_copy(zbuf,
                                acc.at[pl.ds(rowbase + i * ZROWS, ZROWS)])
                return _
            lax.fori_loop(0, ROWS_PER_TILE // ZROWS, zacc, None)

            plsc.subcore_barrier()

            def step(it, _):
                handles = []
                for b in range(nbuf):
                    (ids_s, ids_d, gsx, gdx, lidx, srows, drows, msg,
                     gsem, isem) = slots[b]
                    estart = ebase + (it * nbuf + b) * EB
                    pltpu.sync_copy(srcs.at[pl.ds(estart, EB)], ids_s)
                    pltpu.sync_copy(dsts.at[pl.ds(estart, EB)], ids_d)
                    cp1 = pltpu.async_copy(tabs.at[ids_s], srows, gsem)
                    cp2 = pltpu.async_copy(tabd.at[ids_d], drows, isem)
                    for g in range(EB // 16):
                        d16 = ids_d[pl.ds(16 * g, 16)]
                        dl = d16 - base
                        own = (dl >= 0) & (dl < CHUNK)
                        lidx[pl.ds(16 * g, 16)] = jnp.where(own, dl, CHUNK)
                    handles.append((cp1, cp2))
                for b in range(nbuf):
                    cp1, cp2 = handles[b]
                    cp1.wait()
                    cp2.wait()
                    compute_and_scatter(b)
                return _
            lax.fori_loop(0, nsteps, step, None)

            plsc.subcore_barrier()
            pltpu.sync_copy(acc.at[pl.ds(rowbase, ROWS_PER_TILE)],
                            out_hbm.at[slab, pl.ds(rowbase, ROWS_PER_TILE)])

    return edge_pass


_edge_pass_cache = {}


def _edge_pass(width, nheads, nbuf):
    key = (width, nheads)
    if key not in _edge_pass_cache:
        _edge_pass_cache[key] = _make_edge_pass(width, nheads, nbuf)
    return _edge_pass_cache[key]


def _dense1_body(x_ref, w_ref, as_ref, ad_ref, ts_ref, td_ref):
    h = jnp.dot(x_ref[...], w_ref[...], preferred_element_type=jnp.float32)
    asv = as_ref[0:1, :]
    adv = ad_ref[0:1, :]
    acols = []
    dcols = []
    for hh in range(HEADS):
        sl = slice(16 * hh, 16 * (hh + 1))
        acols.append(jnp.sum(h[:, sl] * asv[:, sl], axis=1, keepdims=True))
        dcols.append(jnp.sum(h[:, sl] * adv[:, sl], axis=1, keepdims=True))
    z12 = jnp.zeros((BT, 12), jnp.float32)
    ts_ref[...] = jnp.concatenate([h] + acols + [z12], axis=1)
    td_ref[...] = jnp.concatenate(dcols + [z12], axis=1)


def _dense2_body(un_ref, den_ref, b1_ref, w2_ref, as_ref, ad_ref,
                 ts_ref, td_ref):
    den = den_ref[...]
    rep = jnp.concatenate(
        [jnp.broadcast_to(den[:, h:h + 1], (BT, 16)) for h in range(HEADS)],
        axis=1)
    hin = un_ref[...] / rep + b1_ref[0:1, :]
    hin = jnp.where(hin > 0, hin, jnp.exp(jnp.minimum(hin, 0.0)) - 1.0)
    h2 = jnp.dot(hin, w2_ref[...], preferred_element_type=jnp.float32)
    a2s = jnp.sum(h2 * as_ref[0:1, :], axis=1, keepdims=True)
    a2d = jnp.sum(h2 * ad_ref[0:1, :], axis=1, keepdims=True)
    ones = jnp.ones((BT, 1), jnp.float32)
    ts_ref[...] = jnp.concatenate(
        [h2[:, 0:6], ones, a2s, jnp.zeros((BT, 8), jnp.float32)], axis=1)
    td_ref[...] = jnp.concatenate(
        [a2d, jnp.zeros((BT, 15), jnp.float32)], axis=1)


def _final_body(y_ref, b2_ref, o_ref):
    y = y_ref[...]
    den = jnp.broadcast_to(y[:, 6:7], (BT, 6))
    r = y[:, 0:6] / den + b2_ref[0:1, 0:6]
    o_ref[...] = 1.0 / (1.0 + jnp.exp(-r))


def kernel(x, edge_index, W1, att_src1, att_dst1, b1, W2, att_src2,
           att_dst2, b2):
    f32 = jnp.float32
    ei = edge_index.astype(jnp.int32)
    loop = jnp.arange(N, dtype=jnp.int32)
    pad = EP - (E + N)
    srcs = jnp.concatenate([ei[0], loop, jnp.zeros((pad,), jnp.int32)])
    dsts = jnp.concatenate([ei[1], loop, jnp.full((pad,), N, jnp.int32)])

    xp = jnp.pad(x.astype(f32), ((0, 0), (0, 2)))
    w1p = jnp.pad(W1.astype(f32), ((0, 2), (0, 0)))
    as1 = jnp.pad(att_src1.astype(f32).reshape(1, 64), ((0, 7), (0, 0)))
    ad1 = jnp.pad(att_dst1.astype(f32).reshape(1, 64), ((0, 7), (0, 0)))

    tabs1, tabd1 = pl.pallas_call(
        _dense1_body,
        grid=(GRID,),
        in_specs=[
            pl.BlockSpec((BT, 8), lambda i: (i, 0)),
            pl.BlockSpec((8, 64), lambda i: (0, 0)),
            pl.BlockSpec((8, 64), lambda i: (0, 0)),
            pl.BlockSpec((8, 64), lambda i: (0, 0)),
        ],
        out_specs=[
            pl.BlockSpec((BT, 80), lambda i: (i, 0)),
            pl.BlockSpec((BT, 16), lambda i: (i, 0)),
        ],
        out_shape=[
            jax.ShapeDtypeStruct((N, 80), f32),
            jax.ShapeDtypeStruct((N, 16), f32),
        ],
    )(xp, w1p, as1, ad1)

    acc1 = _edge_pass(80, 4, 2)(
        tabs1, jnp.pad(tabd1, ((0, 16), (0, 0))), srcs, dsts)
    part1 = jnp.concatenate(
        [acc1[k, :CHUNK] for k in range(2 * NRANGE)], axis=0)
    un1 = part1[:, 0:64]
    den1 = part1[:, 64:68]

    b1p = jnp.pad(b1.astype(f32).reshape(1, 64), ((0, 7), (0, 0)))
    w2p = jnp.pad(W2.astype(f32), ((0, 0), (0, 16 - OUT_DIM)))
    as2 = jnp.pad(att_src2.astype(f32).reshape(1, OUT_DIM),
                  ((0, 7), (0, 16 - OUT_DIM)))
    ad2 = jnp.pad(att_dst2.astype(f32).reshape(1, OUT_DIM),
                  ((0, 7), (0, 16 - OUT_DIM)))

    tabs2, tabd2 = pl.pallas_call(
        _dense2_body,
        grid=(GRID,),
        in_specs=[
            pl.BlockSpec((BT, 64), lambda i: (i, 0)),
            pl.BlockSpec((BT, 4), lambda i: (i, 0)),
            pl.BlockSpec((8, 64), lambda i: (0, 0)),
            pl.BlockSpec((64, 16), lambda i: (0, 0)),
            pl.BlockSpec((8, 16), lambda i: (0, 0)),
            pl.BlockSpec((8, 16), lambda i: (0, 0)),
        ],
        out_specs=[
            pl.BlockSpec((BT, 16), lambda i: (i, 0)),
            pl.BlockSpec((BT, 16), lambda i: (i, 0)),
        ],
        out_shape=[
            jax.ShapeDtypeStruct((N, 16), f32),
            jax.ShapeDtypeStruct((N, 16), f32),
        ],
    )(un1, den1, b1p, w2p, as2, ad2)

    acc2 = _edge_pass(16, 1, 2)(
        tabs2, jnp.pad(tabd2, ((0, 16), (0, 0))), srcs, dsts)
    y = jnp.concatenate(
        [acc2[k, :CHUNK] for k in range(2 * NRANGE)], axis=0)

    b2p = jnp.pad(b2.astype(f32).reshape(1, OUT_DIM),
                  ((0, 7), (0, 16 - OUT_DIM)))
    out = pl.pallas_call(
        _final_body,
        grid=(GRID,),
        in_specs=[
            pl.BlockSpec((BT, 16), lambda i: (i, 0)),
            pl.BlockSpec((8, 16), lambda i: (0, 0)),
        ],
        out_specs=pl.BlockSpec((BT, OUT_DIM), lambda i: (i, 0)),
        out_shape=jax.ShapeDtypeStruct((N, OUT_DIM), f32),
    )(y, b2p)
    return out
